# baseline (speedup 1.0000x reference)
import jax
import jax.numpy as jnp
from jax.experimental import pallas as pl

_K = 8192
_C = 256
_BETA = 0.25
_TT = 256


def _vq_body(a2_ref, z_ref, emb_ref, b2_ref, idx_ref, zq_ref, loss_ref):
    i = pl.program_id(0)
    z_tile = z_ref[...]
    a2 = a2_ref[...]
    b2 = b2_ref[...]

    mm = jax.lax.dot_general(
        z_tile.astype(jnp.bfloat16), emb_ref[...].astype(jnp.bfloat16),
        dimension_numbers=(((1,), (1,)), ((), ())),
        preferred_element_type=jnp.float32,
    )
    d2 = (a2 - 2.0 * mm) + b2
    d = jnp.sqrt(jnp.maximum(d2, 0.0))
    idx = jnp.argmin(d, axis=1).astype(jnp.int32)
    idx2 = idx[:, None]
    idx_ref[...] = idx2

    onehot = (jax.lax.broadcasted_iota(jnp.int32, (_TT, _K), 1)
              == idx2).astype(jnp.float32)
    zq = jax.lax.dot_general(
        onehot, emb_ref[...],
        dimension_numbers=(((1,), (0,)), ((), ())),
        preferred_element_type=jnp.float32,
    )
    zq_ref[...] = zq

    diff = zq - z_tile
    part = jnp.sum(diff * diff).reshape(1, 1)

    @pl.when(i == 0)
    def _():
        loss_ref[...] = part

    @pl.when(i != 0)
    def _():
        loss_ref[...] += part


@jax.jit
def _vq(z, emb):
    B, C, H, W = z.shape
    N = B * H * W
    z_flat = jnp.transpose(z, (0, 2, 3, 1)).reshape(N, C)
    a2 = jnp.sum(z_flat * z_flat, axis=1, keepdims=True)
    b2 = jnp.sum(emb * emb, axis=1)[None, :]

    grid = (N // _TT,)
    idx, zq, loss_sum = pl.pallas_call(
        _vq_body,
        grid=grid,
        in_specs=[
            pl.BlockSpec((_TT, 1), lambda i: (i, 0)),
            pl.BlockSpec((_TT, C), lambda i: (i, 0)),
            pl.BlockSpec((_K, C), lambda i: (0, 0)),
            pl.BlockSpec((1, _K), lambda i: (0, 0)),
        ],
        out_specs=[
            pl.BlockSpec((_TT, 1), lambda i: (i, 0)),
            pl.BlockSpec((_TT, C), lambda i: (i, 0)),
            pl.BlockSpec((1, 1), lambda i: (0, 0)),
        ],
        out_shape=[
            jax.ShapeDtypeStruct((N, 1), jnp.int32),
            jax.ShapeDtypeStruct((N, C), jnp.float32),
            jax.ShapeDtypeStruct((1, 1), jnp.float32),
        ],
    )(a2, z_flat, emb, b2)

    encoding_indices = idx.reshape(N)
    z_q = jnp.transpose(zq.reshape(B, H, W, C), (0, 3, 1, 2))
    n = jnp.float32(N * C)
    m = loss_sum[0, 0] / n
    return z_q, encoding_indices, m + _BETA * m


def kernel(z, emb):
    return _vq(z, emb)


# speedup vs baseline: 1.0382x; 1.0382x over previous
import jax
import jax.numpy as jnp
from jax.experimental import pallas as pl

_K = 8192
_C = 256
_BETA = 0.25
_TT = 512


def _vq_body(a2_ref, z_ref, emb_ref, b2_ref, idx_ref, zq_ref, loss_ref):
    i = pl.program_id(0)
    z_tile = z_ref[...]
    a2 = a2_ref[...]
    b2 = b2_ref[...]

    mm = jax.lax.dot_general(
        z_tile.astype(jnp.bfloat16), emb_ref[...].astype(jnp.bfloat16),
        dimension_numbers=(((1,), (1,)), ((), ())),
        preferred_element_type=jnp.float32,
    )
    d2 = (a2 - 2.0 * mm) + b2
    d = jnp.sqrt(jnp.maximum(d2, 0.0))
    idx = jnp.argmin(d, axis=1).astype(jnp.int32)
    idx2 = idx[:, None]
    idx_ref[...] = idx2

    onehot = (jax.lax.broadcasted_iota(jnp.int32, (_TT, _K), 1)
              == idx2).astype(jnp.float32)
    zq = jax.lax.dot_general(
        onehot, emb_ref[...],
        dimension_numbers=(((1,), (0,)), ((), ())),
        preferred_element_type=jnp.float32,
    )
    zq_ref[...] = zq

    diff = zq - z_tile
    part = jnp.sum(diff * diff).reshape(1, 1)

    @pl.when(i == 0)
    def _():
        loss_ref[...] = part

    @pl.when(i != 0)
    def _():
        loss_ref[...] += part


@jax.jit
def _vq(z, emb):
    B, C, H, W = z.shape
    N = B * H * W
    z_flat = jnp.transpose(z, (0, 2, 3, 1)).reshape(N, C)
    a2 = jnp.sum(z_flat * z_flat, axis=1, keepdims=True)
    b2 = jnp.sum(emb * emb, axis=1)[None, :]

    grid = (N // _TT,)
    idx, zq, loss_sum = pl.pallas_call(
        _vq_body,
        grid=grid,
        in_specs=[
            pl.BlockSpec((_TT, 1), lambda i: (i, 0)),
            pl.BlockSpec((_TT, C), lambda i: (i, 0)),
            pl.BlockSpec((_K, C), lambda i: (0, 0)),
            pl.BlockSpec((1, _K), lambda i: (0, 0)),
        ],
        out_specs=[
            pl.BlockSpec((_TT, 1), lambda i: (i, 0)),
            pl.BlockSpec((_TT, C), lambda i: (i, 0)),
            pl.BlockSpec((1, 1), lambda i: (0, 0)),
        ],
        out_shape=[
            jax.ShapeDtypeStruct((N, 1), jnp.int32),
            jax.ShapeDtypeStruct((N, C), jnp.float32),
            jax.ShapeDtypeStruct((1, 1), jnp.float32),
        ],
    )(a2, z_flat, emb, b2)

    encoding_indices = idx.reshape(N)
    z_q = jnp.transpose(zq.reshape(B, H, W, C), (0, 3, 1, 2))
    n = jnp.float32(N * C)
    m = loss_sum[0, 0] / n
    return z_q, encoding_indices, m + _BETA * m


def kernel(z, emb):
    return _vq(z, emb)
